# Initial kernel scaffold; baseline (speedup 1.0000x reference)
import jax
import jax.numpy as jnp
from jax.experimental import pallas as pl


def kernel(fake_img, real_img):
    raise NotImplementedError("write your pallas kernel here")


# speedup vs baseline: 1.8353x; 1.8353x over previous
import jax
import jax.numpy as jnp
from jax import lax
from jax.experimental import pallas as pl

_H = 512
_W = 512


def _bbox_from_bool(mask_bool):
    row_has = jnp.max(mask_bool.astype(jnp.float32), axis=1, keepdims=True)
    col_has = jnp.max(mask_bool.astype(jnp.float32), axis=0, keepdims=True)
    idx_r = lax.broadcasted_iota(jnp.int32, (_H, 1), 0).astype(jnp.float32)
    idx_c = lax.broadcasted_iota(jnp.int32, (1, _W), 1).astype(jnp.float32)
    big = 1e9
    rp = row_has > 0.5
    cp = col_has > 0.5
    r0 = jnp.min(jnp.where(rp, idx_r, big))
    r1 = jnp.max(jnp.where(rp, idx_r, -1.0))
    c0 = jnp.min(jnp.where(cp, idx_c, big))
    c1 = jnp.max(jnp.where(cp, idx_c, -1.0))
    has_r = jnp.max(row_has) > 0.5
    has_c = jnp.max(col_has) > 0.5
    r0 = jnp.where(has_r, r0, 0.0)
    r1 = jnp.where(has_r, r1, _H - 1.0)
    c0 = jnp.where(has_c, c0, 0.0)
    c1 = jnp.where(has_c, c1, _W - 1.0)
    return r0, c0, r1, c1


def _area(r0, c0, r1, c1):
    w = r1 - r0
    h = c1 - c0
    deg = jnp.logical_or(w == 0.0, h == 0.0)
    return jnp.where(deg, (w + 1.0) * (h + 1.0), w * h)


def _giou_dice_kernel(f_ref, r_ref, out_ref):
    f = f_ref[0, 0, :, :]
    r = r_ref[0, 0, :, :]
    fmin = jnp.min(f)
    fmax = jnp.max(f)
    fn = (f - fmin) / (fmax - fmin)
    mb = fn > 0.5
    m = mb.astype(jnp.float32)

    pr0, pc0, pr1, pc1 = _bbox_from_bool(mb)
    gr0, gc0, gr1, gc1 = _bbox_from_bool(r > 0)

    area_p = _area(pr0, pc0, pr1, pc1)
    area_gt = _area(gr0, gc0, gr1, gc1)

    xI1 = jnp.maximum(pr0, gr0)
    xI2 = jnp.minimum(pr1, gr1)
    yI1 = jnp.maximum(pc0, gc0)
    yI2 = jnp.minimum(pc1, gc1)
    inter = jnp.maximum(yI2 - yI1, 0.0) * jnp.maximum(xI2 - xI1, 0.0)

    xC1 = jnp.minimum(pr0, gr0)
    xC2 = jnp.maximum(pr1, gr1)
    yC1 = jnp.minimum(pc0, gc0)
    yC2 = jnp.maximum(pc1, gc1)
    c_area = (xC2 - xC1) * (yC2 - yC1)

    union = area_p + area_gt - inter
    iou = inter / union
    giou = iou - (c_area - union) / c_area

    smooth = 1.0
    s_mr = jnp.sum(m * r)
    s_m = jnp.sum(m)
    s_r = jnp.sum(r)
    dice = (2.0 * s_mr + smooth) / (s_m + s_r + smooth)

    row_idx = lax.broadcasted_iota(jnp.int32, (8, 128), 0)
    vals = jnp.where(row_idx == 0, giou,
                     jnp.where(row_idx == 1, dice, 1.0 - giou))
    out_ref[0] = vals


def kernel(fake_img, real_img):
    out = pl.pallas_call(
        _giou_dice_kernel,
        grid=(16,),
        in_specs=[
            pl.BlockSpec((1, 1, _H, _W), lambda i: (i, 0, 0, 0)),
            pl.BlockSpec((1, 1, _H, _W), lambda i: (i, 0, 0, 0)),
        ],
        out_specs=pl.BlockSpec((1, 8, 128), lambda i: (i, 0, 0)),
        out_shape=jax.ShapeDtypeStruct((16, 8, 128), jnp.float32),
    )(fake_img, real_img)
    giou = out[:, 0, 0][None, :]
    dice = out[:, 1, 0][None, :]
    loss_giou = out[:, 2, 0][None, :]
    threshold = jnp.full((1, 16), 0.5, dtype=jnp.float32)
    return (loss_giou, giou, threshold, dice)
